# baseline (speedup 1.0000x reference)
import functools
import jax
import jax.numpy as jnp
from jax import lax
from jax.experimental import pallas as pl
from jax.experimental.pallas import tpu as pltpu
from jax.experimental.pallas import tpu_sc as plsc

_G = 640
_NW = 32
_L = 16
_VPARTS = (48, 48, 21)


def _vec(val, dtype=jnp.float32):
    return jnp.zeros((_L,), dtype) + val


def kernel(pred_obj_logits, pred_verb_logits, pred_sub_boxes, pred_obj_boxes, target_sizes):
    B, Q, C = pred_obj_logits.shape
    V = pred_verb_logits.shape[-1]
    BQ = B * Q
    n_chunks = BQ // _G

    obj_t = jnp.transpose(pred_obj_logits, (2, 0, 1)).reshape(C, BQ)
    verb_t = jnp.transpose(pred_verb_logits, (2, 0, 1)).reshape(V, BQ)
    sub_t = jnp.transpose(pred_sub_boxes, (2, 0, 1)).reshape(4, BQ)
    objb_t = jnp.transpose(pred_obj_boxes, (2, 0, 1)).reshape(4, BQ)

    img_h = target_sizes[:, 0].astype(jnp.float32)
    img_w = target_sizes[:, 1].astype(jnp.float32)
    scw = jnp.repeat(img_w, Q)
    sch = jnp.repeat(img_h, Q)

    mesh = plsc.VectorSubcoreMesh(core_axis_name="c", subcore_axis_name="s")

    @functools.partial(
        pl.kernel, mesh=mesh,
        out_type=[
            jax.ShapeDtypeStruct((V, BQ), jnp.float32),
            jax.ShapeDtypeStruct((BQ,), jnp.int32),
            jax.ShapeDtypeStruct((BQ,), jnp.float32),
            jax.ShapeDtypeStruct((4, BQ), jnp.float32),
            jax.ShapeDtypeStruct((4, BQ), jnp.float32),
        ],
        scratch_types=[
            pltpu.VMEM((C, _G), jnp.float32),
            pltpu.VMEM((_VPARTS[0], _G), jnp.float32),
            pltpu.VMEM((_VPARTS[2], _G), jnp.float32),
            pltpu.VMEM((_G,), jnp.float32),
            pltpu.VMEM((_G,), jnp.int32),
            pltpu.VMEM((_G,), jnp.float32),
            pltpu.VMEM((_G,), jnp.float32),
            pltpu.VMEM((4, _G), jnp.float32),
        ],
    )
    def sc_k(obj_hbm, verb_hbm, sub_hbm, objb_hbm, scw_hbm, sch_hbm,
             vs_hbm, lab_hbm, sc_hbm, subo_hbm, objo_hbm,
             obuf, vbuf_a, vbuf_b, sc_buf, lab_buf, scw_v, sch_v, bbuf):
        wid = lax.axis_index("s") * 2 + lax.axis_index("c")

        def process(off):
            pltpu.sync_copy(obj_hbm.at[:, pl.ds(off, _G)], obuf)

            def vloop_a(v, carry):
                sl = pl.ds(v * _L, _L)

                def pass1(ci, acc):
                    m, lab = acc
                    x = obuf[ci, sl]
                    upd = x > m
                    return (jnp.where(upd, x, m),
                            jnp.where(upd, _vec(ci, jnp.int32), lab))

                m, lab = lax.fori_loop(0, C - 1, pass1,
                                       (_vec(-1e30), _vec(0, jnp.int32)))

                def pass2(ci, s):
                    return s + jnp.exp(obuf[ci, sl] - m)

                s = lax.fori_loop(0, C, pass2, _vec(0.0))
                sc_buf[sl] = 1.0 / s
                lab_buf[sl] = lab
                return carry

            lax.fori_loop(0, _G // _L, vloop_a, 0)
            pltpu.sync_copy(sc_buf, sc_hbm.at[pl.ds(off, _G)])
            pltpu.sync_copy(lab_buf, lab_hbm.at[pl.ds(off, _G)])

            r0 = 0
            for rows in _VPARTS:
                vb = vbuf_a if rows == _VPARTS[0] else vbuf_b
                pltpu.sync_copy(
                    verb_hbm.at[pl.ds(r0, rows), pl.ds(off, _G)], vb)
                def vloop_b(v, carry):
                    sl = pl.ds(v * _L, _L)
                    score = sc_buf[sl]

                    def vrow(ri, acc):
                        x = vb[ri, sl]
                        vb[ri, sl] = score / (1.0 + jnp.exp(-x))
                        return acc

                    lax.fori_loop(0, rows, vrow, 0)
                    return carry

                lax.fori_loop(0, _G // _L, vloop_b, 0)
                pltpu.sync_copy(
                    vb, vs_hbm.at[pl.ds(r0, rows), pl.ds(off, _G)])
                r0 += rows

            pltpu.sync_copy(scw_hbm.at[pl.ds(off, _G)], scw_v)
            pltpu.sync_copy(sch_hbm.at[pl.ds(off, _G)], sch_v)
            for src, dst in ((sub_hbm, subo_hbm), (objb_hbm, objo_hbm)):
                pltpu.sync_copy(src.at[:, pl.ds(off, _G)], bbuf)
                def vloop_c(v, carry):
                    sl = pl.ds(v * _L, _L)
                    cx, cy = bbuf[0, sl], bbuf[1, sl]
                    hw, hh = bbuf[2, sl] * 0.5, bbuf[3, sl] * 0.5
                    w_s, h_s = scw_v[sl], sch_v[sl]
                    bbuf[0, sl] = (cx - hw) * w_s
                    bbuf[1, sl] = (cy - hh) * h_s
                    bbuf[2, sl] = (cx + hw) * w_s
                    bbuf[3, sl] = (cy + hh) * h_s
                    return carry

                lax.fori_loop(0, _G // _L, vloop_c, 0)
                pltpu.sync_copy(bbuf, dst.at[:, pl.ds(off, _G)])

        for k in range(-(-n_chunks // _NW)):
            cidx = k * _NW + wid

            @pl.when(cidx < n_chunks)
            def _():
                process(cidx * _G)

    vs_t, lab_t, sc_t, sub_o, obj_o = sc_k(
        obj_t, verb_t, sub_t, objb_t, scw, sch)

    vs = jnp.transpose(vs_t.reshape(V, B, Q), (1, 2, 0))
    labels = jnp.concatenate(
        [jnp.zeros((B, Q), jnp.int32), lab_t.reshape(B, Q)], axis=1)
    obj_scores = sc_t.reshape(B, Q)
    sub_p = jnp.transpose(sub_o.reshape(4, B, Q), (1, 2, 0))
    obj_p = jnp.transpose(obj_o.reshape(4, B, Q), (1, 2, 0))
    boxes = jnp.concatenate([sub_p, obj_p], axis=1)
    ids = jnp.arange(2 * Q)
    return (labels, boxes, vs, pred_verb_logits, ids[:Q], ids[Q:], obj_scores)


# speedup vs baseline: 3.3534x; 3.3534x over previous
import jax
import jax.numpy as jnp
from jax.experimental import pallas as pl
from jax.experimental.pallas import tpu as pltpu

_QB = 4000


def _postproc_body(obj_ref, verb_ref, sub_ref, objb_ref, scale_ref,
                   labels_ref, subo_ref, objo_ref, vs_ref, scores_ref):
    x = obj_ref[0]
    qb, c = x.shape
    col = jax.lax.broadcasted_iota(jnp.int32, x.shape, 1)
    xm = jnp.where(col < c - 1, x, -jnp.inf)
    m_obj = jnp.max(xm, axis=-1, keepdims=True)
    label = jnp.min(jnp.where(xm == m_obj, col, c), axis=-1, keepdims=True)
    score = 1.0 / jnp.sum(jnp.exp(x - m_obj), axis=-1, keepdims=True)

    vs_ref[0] = jax.nn.sigmoid(verb_ref[0]) * score

    scores_ref[0, 0] = score.reshape(8, qb // 8)
    lab = label.reshape(8, qb // 8)
    labels_ref[0, 0, 0] = jnp.zeros_like(lab)
    labels_ref[0, 1, 0] = lab

    scale = scale_ref[...]
    for src, dst in ((sub_ref, subo_ref), (objb_ref, objo_ref)):
        bx = src[...]
        cxy = bx[:, 0:2]
        half = bx[:, 2:4] * 0.5
        dst[...] = jnp.concatenate([cxy - half, cxy + half], axis=1) * scale


def kernel(pred_obj_logits, pred_verb_logits, pred_sub_boxes, pred_obj_boxes, target_sizes):
    B, Q, C = pred_obj_logits.shape
    V = pred_verb_logits.shape[-1]
    nq = Q // _QB
    qs = _QB // 8
    qbb = 1024

    img_h = target_sizes[:, 0].astype(jnp.float32)
    img_w = target_sizes[:, 1].astype(jnp.float32)
    scale = jnp.stack([img_w, img_h, img_w, img_h], axis=1).reshape(B, 4, 1)

    sub_t = jnp.transpose(pred_sub_boxes, (0, 2, 1))
    objb_t = jnp.transpose(pred_obj_boxes, (0, 2, 1))

    lab5, sub_o, obj_o, vs, sc4 = pl.pallas_call(
        _postproc_body,
        grid=(B, nq),
        in_specs=[
            pl.BlockSpec((1, _QB, C), lambda b, q: (b, q, 0)),
            pl.BlockSpec((1, _QB, V), lambda b, q: (b, q, 0)),
            pl.BlockSpec((B, 4, qbb), lambda b, q, n=nq: (0, 0, b * n + q)),
            pl.BlockSpec((B, 4, qbb), lambda b, q, n=nq: (0, 0, b * n + q)),
            pl.BlockSpec((B, 4, 1), lambda b, q: (0, 0, 0)),
        ],
        out_specs=[
            pl.BlockSpec((1, 2, 1, 8, qs), lambda b, q: (b, 0, q, 0, 0)),
            pl.BlockSpec((B, 4, qbb), lambda b, q, n=nq: (0, 0, b * n + q)),
            pl.BlockSpec((B, 4, qbb), lambda b, q, n=nq: (0, 0, b * n + q)),
            pl.BlockSpec((1, _QB, V), lambda b, q: (b, q, 0)),
            pl.BlockSpec((1, 1, 8, qs), lambda b, q: (b, q, 0, 0)),
        ],
        out_shape=[
            jax.ShapeDtypeStruct((B, 2, nq, 8, qs), jnp.int32),
            jax.ShapeDtypeStruct((B, 4, Q), jnp.float32),
            jax.ShapeDtypeStruct((B, 4, Q), jnp.float32),
            jax.ShapeDtypeStruct((B, Q, V), jnp.float32),
            jax.ShapeDtypeStruct((B, nq, 8, qs), jnp.float32),
        ],
        compiler_params=pltpu.CompilerParams(
            dimension_semantics=("parallel", "parallel")),
    )(pred_obj_logits, pred_verb_logits, sub_t, objb_t, scale)

    labels = lab5.reshape(B, 2 * Q)
    boxes = jnp.transpose(jnp.concatenate([sub_o, obj_o], axis=2), (0, 2, 1))
    obj_scores = sc4.reshape(B, Q)
    ids = jnp.arange(2 * Q)
    return (labels, boxes, vs, pred_verb_logits, ids[:Q], ids[Q:], obj_scores)
